# baseline (speedup 1.0000x reference)
import functools

import jax
import jax.numpy as jnp
from jax import lax
from jax.experimental import pallas as pl
from jax.experimental.pallas import tpu as pltpu
from jax.experimental.pallas import tpu_sc as plsc

VOCAB = 1000000
EMBED_DIM = 64
BATCH = 4096
HIST = 200

NUM_ROWS = BATCH * HIST
NC, NS = 2, 16
NW = NC * NS
B_PER_W = NUM_ROWS // NW
IDX_PER_STREAM = 128
CHUNK = 1024
STREAMS_PER_CHUNK = CHUNK // IDX_PER_STREAM
N_CHUNKS = B_PER_W // CHUNK


def _gather_body(idx_hbm, table_hbm, out_hbm, idx_v, rows_v, sem):
    wid = lax.axis_index("s") * NC + lax.axis_index("c")
    base = wid * B_PER_W
    pltpu.sync_copy(idx_hbm.at[pl.ds(base, B_PER_W)], idx_v)

    def chunk_step(g, carry):
        off = g * CHUNK
        descs = []
        for j in range(STREAMS_PER_CHUNK):
            descs.append(pltpu.async_copy(
                table_hbm.at[idx_v.at[pl.ds(off + j * IDX_PER_STREAM,
                                            IDX_PER_STREAM)]],
                rows_v.at[pl.ds(j * IDX_PER_STREAM, IDX_PER_STREAM)],
                sem))
        for d in descs:
            d.wait()
        pltpu.sync_copy(rows_v, out_hbm.at[pl.ds(base + off, CHUNK)])
        return carry

    lax.fori_loop(0, N_CHUNKS, chunk_step, 0)


def kernel(input_ids, table):
    idx_flat = input_ids.reshape(NUM_ROWS).astype(jnp.int32)
    mesh = plsc.VectorSubcoreMesh(core_axis_name="c", subcore_axis_name="s")
    run = functools.partial(
        pl.kernel,
        mesh=mesh,
        out_type=jax.ShapeDtypeStruct((NUM_ROWS, EMBED_DIM), jnp.float32),
        scratch_types=[
            pltpu.VMEM((B_PER_W,), jnp.int32),
            pltpu.VMEM((CHUNK, EMBED_DIM), jnp.float32),
            pltpu.SemaphoreType.DMA,
        ],
        compiler_params=pltpu.CompilerParams(use_tc_tiling_on_sc=False),
    )(_gather_body)
    out = run(idx_flat, table)
    return out.reshape(BATCH, HIST, EMBED_DIM)


# speedup vs baseline: 1.0056x; 1.0056x over previous
import functools

import jax
import jax.numpy as jnp
from jax import lax
from jax.experimental import pallas as pl
from jax.experimental.pallas import tpu as pltpu
from jax.experimental.pallas import tpu_sc as plsc

VOCAB = 1000000
EMBED_DIM = 64
BATCH = 4096
HIST = 200

NUM_ROWS = BATCH * HIST
NC, NS = 2, 16
NW = NC * NS
B_PER_W = NUM_ROWS // NW
IDX_PER_STREAM = 128
CHUNK = 512
STREAMS_PER_CHUNK = CHUNK // IDX_PER_STREAM
NBUF = 2
N_CHUNKS = B_PER_W // CHUNK
N_GROUPS = N_CHUNKS // NBUF


def _gather_body(idx_hbm, table_hbm, out_hbm, idx_v, rows0, rows1,
                 gsem0, gsem1, wsem0, wsem1):
    wid = lax.axis_index("s") * NC + lax.axis_index("c")
    base = wid * B_PER_W
    bufs = (rows0, rows1)
    gsems = (gsem0, gsem1)
    wsems = (wsem0, wsem1)
    pltpu.sync_copy(idx_hbm.at[pl.ds(base, B_PER_W)], idx_v)

    def fire_gathers(chunk_id, buf, gsem):
        off = chunk_id * CHUNK
        for j in range(STREAMS_PER_CHUNK):
            pltpu.async_copy(
                table_hbm.at[idx_v.at[pl.ds(off + j * IDX_PER_STREAM,
                                            IDX_PER_STREAM)]],
                buf.at[pl.ds(j * IDX_PER_STREAM, IDX_PER_STREAM)],
                gsem)

    def drain_gathers(buf, gsem):
        for j in range(STREAMS_PER_CHUNK):
            pltpu.make_async_copy(
                table_hbm.at[pl.ds(0, IDX_PER_STREAM)],
                buf.at[pl.ds(j * IDX_PER_STREAM, IDX_PER_STREAM)],
                gsem).wait()

    for b in range(NBUF):
        fire_gathers(b, bufs[b], gsems[b])

    def group_step(k, carry):
        c0 = k * NBUF
        for b in range(NBUF):
            drain_gathers(bufs[b], gsems[b])
            pltpu.async_copy(bufs[b],
                             out_hbm.at[pl.ds(base + (c0 + b) * CHUNK, CHUNK)],
                             wsems[b])
        for b in range(NBUF):
            pltpu.make_async_copy(
                bufs[b], out_hbm.at[pl.ds(base, CHUNK)], wsems[b]).wait()

            @pl.when(k < N_GROUPS - 1)
            def _():
                fire_gathers(c0 + NBUF + b, bufs[b], gsems[b])
        return carry

    lax.fori_loop(0, N_GROUPS, group_step, 0)


def kernel(input_ids, table):
    idx_flat = input_ids.reshape(NUM_ROWS).astype(jnp.int32)
    mesh = plsc.VectorSubcoreMesh(core_axis_name="c", subcore_axis_name="s")
    run = functools.partial(
        pl.kernel,
        mesh=mesh,
        out_type=jax.ShapeDtypeStruct((NUM_ROWS, EMBED_DIM), jnp.float32),
        scratch_types=[
            pltpu.VMEM((B_PER_W,), jnp.int32),
            pltpu.VMEM((CHUNK, EMBED_DIM), jnp.float32),
            pltpu.VMEM((CHUNK, EMBED_DIM), jnp.float32),
            pltpu.SemaphoreType.DMA,
            pltpu.SemaphoreType.DMA,
            pltpu.SemaphoreType.DMA,
            pltpu.SemaphoreType.DMA,
        ],
        compiler_params=pltpu.CompilerParams(use_tc_tiling_on_sc=False),
    )(_gather_body)
    out = run(idx_flat, table)
    return out.reshape(BATCH, HIST, EMBED_DIM)


# speedup vs baseline: 1.0065x; 1.0009x over previous
import functools

import jax
import jax.numpy as jnp
from jax import lax
from jax.experimental import pallas as pl
from jax.experimental.pallas import tpu as pltpu
from jax.experimental.pallas import tpu_sc as plsc

VOCAB = 1000000
EMBED_DIM = 64
BATCH = 4096
HIST = 200

NC, NS = 2, 16
NW = NC * NS
IDX_PER_STREAM = 128
CHUNK = 512
STREAMS_PER_CHUNK = CHUNK // IDX_PER_STREAM
BLOCKS_PER_T = BATCH // CHUNK
N_CHUNKS = HIST * BLOCKS_PER_T
CH_PER_W = N_CHUNKS // NW
NBUF = 2
N_GROUPS = CH_PER_W // NBUF


def _gather_body(ids_t_hbm, table_hbm, out_hbm, idx0, idx1, rows0, rows1,
                 isem0, isem1, gsem0, gsem1, wsem0, wsem1):
    wid = lax.axis_index("s") * NC + lax.axis_index("c")
    c_base = wid * CH_PER_W
    idxs = (idx0, idx1)
    bufs = (rows0, rows1)
    isems = (isem0, isem1)
    gsems = (gsem0, gsem1)
    wsems = (wsem0, wsem1)

    def fire_idx(c, b):
        t = c // BLOCKS_PER_T
        b0 = (c % BLOCKS_PER_T) * CHUNK
        pltpu.async_copy(ids_t_hbm.at[t, pl.ds(b0, CHUNK)], idxs[b], isems[b])

    def fire_gathers(b):
        for j in range(STREAMS_PER_CHUNK):
            pltpu.async_copy(
                table_hbm.at[idxs[b].at[pl.ds(j * IDX_PER_STREAM,
                                              IDX_PER_STREAM)]],
                bufs[b].at[pl.ds(j * IDX_PER_STREAM, IDX_PER_STREAM)],
                gsems[b])

    def drain_gathers(b):
        for j in range(STREAMS_PER_CHUNK):
            pltpu.make_async_copy(
                table_hbm.at[pl.ds(0, IDX_PER_STREAM)],
                bufs[b].at[pl.ds(j * IDX_PER_STREAM, IDX_PER_STREAM)],
                gsems[b]).wait()

    def fire_wb(c, b):
        t = c // BLOCKS_PER_T
        b0 = (c % BLOCKS_PER_T) * CHUNK
        pltpu.async_copy(bufs[b], out_hbm.at[pl.ds(b0, CHUNK), t], wsems[b])

    for b in range(NBUF):
        fire_idx(c_base + b, b)
    for b in range(NBUF):
        pltpu.make_async_copy(ids_t_hbm.at[0, pl.ds(0, CHUNK)], idxs[b],
                              isems[b]).wait()
        fire_gathers(b)

    def group_step(k, carry):
        for b in range(NBUF):
            c = c_base + k * NBUF + b
            drain_gathers(b)
            fire_wb(c, b)

            @pl.when(k < N_GROUPS - 1)
            def _():
                fire_idx(c + NBUF, b)
        for b in range(NBUF):
            c = c_base + k * NBUF + b
            pltpu.make_async_copy(
                bufs[b], out_hbm.at[pl.ds(0, CHUNK), 0], wsems[b]).wait()

            @pl.when(k < N_GROUPS - 1)
            def _():
                pltpu.make_async_copy(ids_t_hbm.at[0, pl.ds(0, CHUNK)],
                                      idxs[b], isems[b]).wait()
                fire_gathers(b)
        return carry

    lax.fori_loop(0, N_GROUPS, group_step, 0)


def kernel(input_ids, table):
    ids_t = input_ids.T.astype(jnp.int32)
    mesh = plsc.VectorSubcoreMesh(core_axis_name="c", subcore_axis_name="s")
    run = functools.partial(
        pl.kernel,
        mesh=mesh,
        out_type=jax.ShapeDtypeStruct((BATCH, HIST, EMBED_DIM), jnp.float32),
        scratch_types=[
            pltpu.VMEM((CHUNK,), jnp.int32),
            pltpu.VMEM((CHUNK,), jnp.int32),
            pltpu.VMEM((CHUNK, EMBED_DIM), jnp.float32),
            pltpu.VMEM((CHUNK, EMBED_DIM), jnp.float32),
            pltpu.SemaphoreType.DMA,
            pltpu.SemaphoreType.DMA,
            pltpu.SemaphoreType.DMA,
            pltpu.SemaphoreType.DMA,
            pltpu.SemaphoreType.DMA,
            pltpu.SemaphoreType.DMA,
        ],
        compiler_params=pltpu.CompilerParams(use_tc_tiling_on_sc=False),
    )(_gather_body)
    return run(ids_t, table)
